# baseline (speedup 1.0000x reference)
import math

import jax
import jax.numpy as jnp
import numpy as np
from jax.experimental import pallas as pl
from jax.experimental.pallas import tpu as pltpu

D_POSITION = 1024
D_CONTENT = 4096
NUM_TILES = 64
MAXROWS = 512
NBUF = 3


def _sinusoidal_pe(seq_len, d_model):
    pe = np.zeros((seq_len, d_model), dtype=np.float32)
    position = np.arange(0, seq_len, dtype=np.float32)[:, None]
    div_term = np.exp(
        np.arange(0, d_model, 2, dtype=np.float32) * (-math.log(10000.0) / d_model)
    )
    pe[:, 0::2] = np.sin(position * div_term)
    pe[:, 1::2] = np.cos(position * div_term)
    return pe


def _dot(a, b):
    return jax.lax.dot_general(
        a, b, (((1,), (0,)), ((), ())), preferred_element_type=jnp.float32)


def _chunk_plan(rows_total, seq):
    starts, lens = [], []
    st = 0
    while rows_total - st > MAXROWS:
        starts.append(st)
        lens.append(MAXROWS)
        st += MAXROWS
    rem = rows_total - st
    a = (3 * rem // 4) // 128 * 128
    if 0 < a < rem:
        starts.append(st)
        lens.append(a)
        st += a
        rem -= a
    starts.append(st)
    lens.append(rem)
    return starts, lens


def _router_body(seq, starts, lens):
    n_chunks = len(starts)

    def body(wts_ref, x_ref, pe_ref, sigp_ref, sigc_ref,
             scores_ref, idx_ref,
             buf_ref, pev_ref, sigpv_ref, sigcv_ref,
             xsem_ref, csem_ref):
        pw = wts_ref[0]
        cw = wts_ref[1]

        def copy(c):
            slot = c % NBUF
            return pltpu.make_async_copy(
                x_ref.at[pl.ds(starts[c], lens[c]), :],
                buf_ref.at[slot, pl.ds(0, lens[c]), :],
                xsem_ref.at[slot])

        consts = [
            pltpu.make_async_copy(sigc_ref, sigcv_ref, csem_ref.at[0]),
            pltpu.make_async_copy(sigp_ref, sigpv_ref, csem_ref.at[1]),
            pltpu.make_async_copy(pe_ref, pev_ref, csem_ref.at[2]),
        ]

        copy(0).start()
        for c in consts:
            c.start()
        for c in range(1, min(NBUF, n_chunks)):
            copy(c).start()

        consts[1].wait()
        consts[2].wait()
        posb = _dot(pev_ref[...], sigpv_ref[...])
        consts[0].wait()

        iota = jax.lax.broadcasted_iota(jnp.int32, (MAXROWS, NUM_TILES), 1)

        for c in range(n_chunks):
            slot = c % NBUF
            st, ln = starts[c], lens[c]
            copy(c).wait()
            content = _dot(buf_ref[slot, pl.ds(0, ln), :], sigcv_ref[...])
            if c + NBUF < n_chunks:
                copy(c + NBUF).start()
            s0 = st % seq
            scores = cw * content + pw * posb[s0:s0 + ln]
            scores_ref[pl.ds(st, ln), :] = scores

            mx = jnp.max(scores, axis=-1, keepdims=True)
            idx = jnp.min(jnp.where(scores == mx, iota[:ln], NUM_TILES), axis=-1)
            idx_ref[pl.ds(st // 128, ln // 128), :] = idx.reshape(ln // 128, 128)

    return body


def kernel(x, positions, signatures, position_weight, content_weight):
    del positions
    batch, seq, _ = x.shape
    rows_total = batch * seq
    starts, lens = _chunk_plan(rows_total, seq)
    pe = jnp.asarray(_sinusoidal_pe(seq, D_POSITION))
    sig_pos = signatures[:, :D_POSITION].T
    sig_con = signatures[:, D_POSITION:].T

    pw = jax.nn.sigmoid(position_weight)
    cw = jax.nn.sigmoid(content_weight)
    total = pw + cw
    wts = jnp.stack([pw / total, cw / total])

    x2 = x.reshape(rows_total, D_CONTENT)

    scores2, idx2 = pl.pallas_call(
        _router_body(seq, starts, lens),
        in_specs=[
            pl.BlockSpec(memory_space=pltpu.SMEM),
            pl.BlockSpec(memory_space=pl.ANY),
            pl.BlockSpec(memory_space=pl.ANY),
            pl.BlockSpec(memory_space=pl.ANY),
            pl.BlockSpec(memory_space=pl.ANY),
        ],
        out_specs=[
            pl.BlockSpec(memory_space=pltpu.VMEM),
            pl.BlockSpec(memory_space=pltpu.VMEM),
        ],
        out_shape=[
            jax.ShapeDtypeStruct((rows_total, NUM_TILES), jnp.float32),
            jax.ShapeDtypeStruct((rows_total // 128, 128), jnp.int32),
        ],
        scratch_shapes=[
            pltpu.VMEM((NBUF, MAXROWS, D_CONTENT), jnp.float32),
            pltpu.VMEM((seq, D_POSITION), jnp.float32),
            pltpu.VMEM((D_POSITION, NUM_TILES), jnp.float32),
            pltpu.VMEM((D_CONTENT, NUM_TILES), jnp.float32),
            pltpu.SemaphoreType.DMA((NBUF,)),
            pltpu.SemaphoreType.DMA((3,)),
        ],
    )(wts, x2, pe, sig_pos, sig_con)

    scores = scores2.reshape(batch, seq, NUM_TILES)
    indices = idx2.reshape(batch, seq)
    return indices, scores


# speedup vs baseline: 1.1784x; 1.1784x over previous
import math

import jax
import jax.numpy as jnp
import numpy as np
from jax.experimental import pallas as pl
from jax.experimental.pallas import tpu as pltpu

D_POSITION = 1024
D_CONTENT = 4096
NUM_TILES = 64
ROWS = 1024


def _sinusoidal_pe(seq_len, d_model):
    pe = np.zeros((seq_len, d_model), dtype=np.float32)
    position = np.arange(0, seq_len, dtype=np.float32)[:, None]
    div_term = np.exp(
        np.arange(0, d_model, 2, dtype=np.float32) * (-math.log(10000.0) / d_model)
    )
    pe[:, 0::2] = np.sin(position * div_term)
    pe[:, 1::2] = np.cos(position * div_term)
    return pe


def _dot(a, b):
    return jax.lax.dot_general(
        a, b, (((1,), (0,)), ((), ())), preferred_element_type=jnp.float32)


def _router_body(seq):
    reps = ROWS // seq

    def body(wts_ref, pe_ref, x_ref, sigp_ref, sigc_ref,
             scores_ref, idx_ref, posb_ref):
        i = pl.program_id(0)
        pw = wts_ref[0]
        cw = wts_ref[1]

        @pl.when(i == 0)
        def _():
            posb_ref[...] = _dot(pe_ref[...], sigp_ref[...])

        content = _dot(x_ref[...], sigc_ref[...])
        posb = jnp.concatenate([posb_ref[...]] * reps, axis=0)
        scores = cw * content + pw * posb
        scores_ref[...] = scores

        mx = jnp.max(scores, axis=-1, keepdims=True)
        iota = jax.lax.broadcasted_iota(jnp.int32, scores.shape, 1)
        idx = jnp.min(jnp.where(scores == mx, iota, NUM_TILES), axis=-1)
        idx_ref[...] = idx.reshape(ROWS // 128, 128)

    return body


def kernel(x, positions, signatures, position_weight, content_weight):
    del positions
    batch, seq, _ = x.shape
    rows_total = batch * seq
    n_steps = rows_total // ROWS
    pe = jnp.asarray(_sinusoidal_pe(seq, D_POSITION))
    sig_pos = signatures[:, :D_POSITION].T
    sig_con = signatures[:, D_POSITION:].T

    pw = jax.nn.sigmoid(position_weight)
    cw = jax.nn.sigmoid(content_weight)
    total = pw + cw
    wts = jnp.stack([pw / total, cw / total])

    x2 = x.reshape(rows_total, D_CONTENT)

    scores2, idx2 = pl.pallas_call(
        _router_body(seq),
        grid=(n_steps,),
        in_specs=[
            pl.BlockSpec(memory_space=pltpu.SMEM),
            pl.BlockSpec((seq, D_POSITION), lambda i: (0, 0)),
            pl.BlockSpec((ROWS, D_CONTENT), lambda i: (i, 0)),
            pl.BlockSpec((D_POSITION, NUM_TILES), lambda i: (0, 0)),
            pl.BlockSpec((D_CONTENT, NUM_TILES), lambda i: (0, 0)),
        ],
        out_specs=[
            pl.BlockSpec((ROWS, NUM_TILES), lambda i: (i, 0)),
            pl.BlockSpec((ROWS // 128, 128), lambda i: (i, 0)),
        ],
        out_shape=[
            jax.ShapeDtypeStruct((rows_total, NUM_TILES), jnp.float32),
            jax.ShapeDtypeStruct((rows_total // 128, 128), jnp.int32),
        ],
        scratch_shapes=[pltpu.VMEM((seq, NUM_TILES), jnp.float32)],
    )(wts, pe, x2, sig_pos, sig_con)

    scores = scores2.reshape(batch, seq, NUM_TILES)
    indices = idx2.reshape(batch, seq)
    return indices, scores
